# baseline (speedup 1.0000x reference)
import jax
import jax.numpy as jnp
from jax import lax
from jax.experimental import pallas as pl
from jax.experimental.pallas import tpu as pltpu
from jax.experimental.pallas import tpu_sc as plsc

N_EXPERTS = 8
TOP_K = 2
D_MODEL = 1024
D_FF = 4096
N_TOKENS = 8192
N_SLOTS = N_TOKENS * TOP_K

B_ROWS = 768
NB = (N_SLOTS + N_EXPERTS * B_ROWS + B_ROWS - 1) // B_ROWS
CAPACITY = NB * B_ROWS
F_TILE = 512
NF = D_FF // F_TILE

SC_CORES = 2
SC_SUBCORES = 16
SC_WORKERS = SC_CORES * SC_SUBCORES
LANES = 16


_META_R = 128
_BE_PAD = 64


def _metadata_body(e_ref, dest_ref, be_ref):
    e = e_ref[...]
    r = lax.broadcasted_iota(jnp.int32, (_META_R, _META_R), 0)
    c = lax.broadcasted_iota(jnp.int32, (_META_R, _META_R), 1)
    sl = (r < c).astype(jnp.float32)
    slt = (r > c).astype(jnp.float32)

    dest = jnp.zeros((_META_R, _META_R), jnp.int32)
    start = jnp.zeros((1, 1), jnp.int32)
    blk_end = jnp.zeros((1, 1), jnp.int32)
    bidx = lax.broadcasted_iota(jnp.int32, (1, _BE_PAD), 1)
    be = jnp.zeros((1, _BE_PAD), jnp.int32)
    for ex in range(N_EXPERTS):
        m = (e == ex).astype(jnp.float32)
        ex_row = jnp.dot(m, sl, preferred_element_type=jnp.float32)
        rowsum = jnp.sum(m, axis=1, keepdims=True)
        rowoff = jnp.dot(slt, rowsum, preferred_element_type=jnp.float32)
        rank = (ex_row + rowoff).astype(jnp.int32)
        cnt = jnp.sum(rowsum).astype(jnp.int32).reshape(1, 1)
        dest = dest + jnp.where(e == ex, start + rank, 0)
        nblk = (cnt + (B_ROWS - 1)) // B_ROWS
        start = start + nblk * B_ROWS
        blk_end = blk_end + nblk
        be = be + (bidx >= blk_end).astype(jnp.int32)
    dest_ref[...] = dest
    be_out = jnp.where(bidx == _BE_PAD - 1, blk_end,
                       jnp.minimum(be, N_EXPERTS - 1))
    be_ref[...] = be_out


def _metadata(eflat2d):
    return pl.pallas_call(
        _metadata_body,
        out_shape=(
            jax.ShapeDtypeStruct((_META_R, _META_R), jnp.int32),
            jax.ShapeDtypeStruct((1, _BE_PAD), jnp.int32),
        ),
    )(eflat2d)


_DISP_CHUNK = 64
_DISP_TOK_PER_W = N_TOKENS // SC_WORKERS


def _dispatch_body(x_hbm, dest_hbm, wts_hbm, xs_hbm, ws_hbm,
                   de_v, do_v, rows_v, we_v, wo_v, sem_s):
    wid = lax.axis_index("s") * SC_CORES + lax.axis_index("c")
    base_t = wid * _DISP_TOK_PER_W

    def chunk(ci, carry):
        t0 = base_t + ci * _DISP_CHUNK
        pltpu.sync_copy(x_hbm.at[pl.ds(t0, _DISP_CHUNK)], rows_v)
        pltpu.sync_copy(dest_hbm.at[pl.ds(t0, _DISP_CHUNK)], de_v)
        pltpu.sync_copy(dest_hbm.at[pl.ds(N_TOKENS + t0, _DISP_CHUNK)], do_v)
        pltpu.sync_copy(wts_hbm.at[pl.ds(t0, _DISP_CHUNK)], we_v)
        pltpu.sync_copy(wts_hbm.at[pl.ds(N_TOKENS + t0, _DISP_CHUNK)], wo_v)
        cp1 = pltpu.async_copy(rows_v, xs_hbm.at[de_v], sem_s)
        cp2 = pltpu.async_copy(rows_v, xs_hbm.at[do_v], sem_s)
        cp3 = pltpu.async_copy(we_v, ws_hbm.at[de_v], sem_s)
        cp4 = pltpu.async_copy(wo_v, ws_hbm.at[do_v], sem_s)
        cp1.wait()
        cp2.wait()
        cp3.wait()
        cp4.wait()
        return carry

    lax.fori_loop(0, _DISP_TOK_PER_W // _DISP_CHUNK, chunk, 0)


def _dispatch(x, dest, wflat):
    mesh = plsc.VectorSubcoreMesh(core_axis_name="c", subcore_axis_name="s")
    return pl.kernel(
        _dispatch_body,
        out_type=(
            jax.ShapeDtypeStruct((CAPACITY, D_MODEL), jnp.float32),
            jax.ShapeDtypeStruct((CAPACITY,), jnp.float32),
        ),
        mesh=mesh,
        scratch_types=[
            pltpu.VMEM((_DISP_CHUNK,), jnp.int32),
            pltpu.VMEM((_DISP_CHUNK,), jnp.int32),
            pltpu.VMEM((_DISP_CHUNK, D_MODEL), jnp.float32),
            pltpu.VMEM((_DISP_CHUNK,), jnp.float32),
            pltpu.VMEM((_DISP_CHUNK,), jnp.float32),
            pltpu.SemaphoreType.DMA,
        ],
    )(x, dest, wflat)


def _ffn_body(be_ref, xs_ref, ws_ref, w1_ref, w2_ref, w3_ref, out_ref):
    b = pl.program_id(0)
    f = pl.program_id(1)
    nv = be_ref[_BE_PAD - 1]

    @pl.when(b < nv)
    def _():
        x = xs_ref[...]
        g = jnp.dot(x, w1_ref[0], preferred_element_type=jnp.float32)
        v = jnp.dot(x, w2_ref[0], preferred_element_type=jnp.float32)
        h = (g * lax.logistic(g)) * v
        p = jnp.dot(h, w3_ref[0], preferred_element_type=jnp.float32)
        p = p * ws_ref[...]

        @pl.when(f == 0)
        def _():
            out_ref[...] = p

        @pl.when(f != 0)
        def _():
            out_ref[...] = out_ref[...] + p


def _serp(b, f):
    return jnp.where(b % 2 == 0, f, NF - 1 - f)


def _fidx(b, f, be_r):
    nv = be_r[_BE_PAD - 1]
    bb = jnp.minimum(b, nv - 1)
    fe = _serp(bb, jnp.where(b < nv, f, NF - 1))
    return be_r[bb], fe


def _ffn(be, xs, ws, w1, w2, w3):
    def w12_map(b, f, be_r):
        e, fe = _fidx(b, f, be_r)
        return (e, 0, fe)

    def w3_map(b, f, be_r):
        e, fe = _fidx(b, f, be_r)
        return (e, fe, 0)

    def xs_map(b, f, be_r):
        return (jnp.minimum(b, be_r[_BE_PAD - 1] - 1), 0)

    grid_spec = pltpu.PrefetchScalarGridSpec(
        num_scalar_prefetch=1,
        grid=(NB, NF),
        in_specs=[
            pl.BlockSpec((B_ROWS, D_MODEL), xs_map),
            pl.BlockSpec((B_ROWS, 1), xs_map),
            pl.BlockSpec((1, D_MODEL, F_TILE), w12_map),
            pl.BlockSpec((1, D_MODEL, F_TILE), w12_map),
            pl.BlockSpec((1, F_TILE, D_MODEL), w3_map),
        ],
        out_specs=pl.BlockSpec((B_ROWS, D_MODEL), lambda b, f, be_r: (b, 0)),
    )
    return pl.pallas_call(
        _ffn_body,
        grid_spec=grid_spec,
        out_shape=jax.ShapeDtypeStruct((CAPACITY, D_MODEL), jnp.float32),
        compiler_params=pltpu.CompilerParams(
            dimension_semantics=("arbitrary", "arbitrary")),
    )(be, xs, ws, w1, w2, w3)


_COMB_CHUNK = 32
_TOK_PER_W = N_TOKENS // SC_WORKERS


def _combine_body(y_hbm, dest_hbm, out_hbm, de_v, do_v, acc_v, ro_v, sem_g):
    wid = lax.axis_index("s") * SC_CORES + lax.axis_index("c")
    base_t = wid * _TOK_PER_W

    def chunk(ci, carry):
        t0 = base_t + ci * _COMB_CHUNK
        pltpu.sync_copy(dest_hbm.at[pl.ds(t0, _COMB_CHUNK)], de_v)
        pltpu.sync_copy(dest_hbm.at[pl.ds(N_TOKENS + t0, _COMB_CHUNK)], do_v)
        cp1 = pltpu.async_copy(y_hbm.at[de_v], acc_v, sem_g)
        cp2 = pltpu.async_copy(y_hbm.at[do_v], ro_v, sem_g)
        cp1.wait()
        cp2.wait()

        def tok(j, c2):
            for l in range(D_MODEL // LANES):
                sld = pl.ds(l * LANES, LANES)
                acc_v[j, sld] = acc_v[j, sld] + ro_v[j, sld]
            return c2

        lax.fori_loop(0, _COMB_CHUNK, tok, 0)
        pltpu.sync_copy(acc_v, out_hbm.at[pl.ds(t0, _COMB_CHUNK)])
        return carry

    lax.fori_loop(0, _TOK_PER_W // _COMB_CHUNK, chunk, 0)


def _combine(y, dest):
    mesh = plsc.VectorSubcoreMesh(core_axis_name="c", subcore_axis_name="s")
    return pl.kernel(
        _combine_body,
        out_type=jax.ShapeDtypeStruct((N_TOKENS, D_MODEL), jnp.float32),
        mesh=mesh,
        scratch_types=[
            pltpu.VMEM((_COMB_CHUNK,), jnp.int32),
            pltpu.VMEM((_COMB_CHUNK,), jnp.int32),
            pltpu.VMEM((_COMB_CHUNK, D_MODEL), jnp.float32),
            pltpu.VMEM((_COMB_CHUNK, D_MODEL), jnp.float32),
            pltpu.SemaphoreType.DMA,
        ],
    )(y, dest)


def kernel(x, expert_indices, expert_weights, w1, w2, w3):
    eflat2d = expert_indices.astype(jnp.int32).T.reshape(_META_R, _META_R)
    dest2d, be_pad = _metadata(eflat2d)
    dest = dest2d.reshape(N_SLOTS)
    be = be_pad[0]
    wflat = expert_weights.T.reshape(N_SLOTS)
    xs, ws = _dispatch(x, dest, wflat)
    yw = _ffn(be, xs, ws.reshape(CAPACITY, 1), w1, w2, w3)
    return _combine(yw, dest)


# speedup vs baseline: 1.1634x; 1.1634x over previous
import jax
import jax.numpy as jnp
from jax import lax
from jax.experimental import pallas as pl
from jax.experimental.pallas import tpu as pltpu
from jax.experimental.pallas import tpu_sc as plsc

N_EXPERTS = 8
TOP_K = 2
D_MODEL = 1024
D_FF = 4096
N_TOKENS = 8192
N_SLOTS = N_TOKENS * TOP_K

B_ROWS = 768
NB = (N_SLOTS + N_EXPERTS * B_ROWS + B_ROWS - 1) // B_ROWS
CAPACITY = NB * B_ROWS
F_TILE = 1024
NF = D_FF // F_TILE

SC_CORES = 2
SC_SUBCORES = 16
SC_WORKERS = SC_CORES * SC_SUBCORES
LANES = 16


_META_R = 128
_BE_PAD = 64


def _metadata_body(e_ref, dest_ref, be_ref):
    e = e_ref[...]
    r = lax.broadcasted_iota(jnp.int32, (_META_R, _META_R), 0)
    c = lax.broadcasted_iota(jnp.int32, (_META_R, _META_R), 1)
    sl = (r < c).astype(jnp.float32)
    slt = (r > c).astype(jnp.float32)

    dest = jnp.zeros((_META_R, _META_R), jnp.int32)
    start = jnp.zeros((1, 1), jnp.int32)
    blk_end = jnp.zeros((1, 1), jnp.int32)
    bidx = lax.broadcasted_iota(jnp.int32, (1, _BE_PAD), 1)
    be = jnp.zeros((1, _BE_PAD), jnp.int32)
    for ex in range(N_EXPERTS):
        m = (e == ex).astype(jnp.float32)
        ex_row = jnp.dot(m, sl, preferred_element_type=jnp.float32)
        rowsum = jnp.sum(m, axis=1, keepdims=True)
        rowoff = jnp.dot(slt, rowsum, preferred_element_type=jnp.float32)
        rank = (ex_row + rowoff).astype(jnp.int32)
        cnt = jnp.sum(rowsum).astype(jnp.int32).reshape(1, 1)
        dest = dest + jnp.where(e == ex, start + rank, 0)
        nblk = (cnt + (B_ROWS - 1)) // B_ROWS
        start = start + nblk * B_ROWS
        blk_end = blk_end + nblk
        be = be + (bidx >= blk_end).astype(jnp.int32)
    dest_ref[...] = dest
    be_out = jnp.where(bidx == _BE_PAD - 1, blk_end,
                       jnp.minimum(be, N_EXPERTS - 1))
    be_ref[...] = be_out


def _metadata(eflat2d):
    return pl.pallas_call(
        _metadata_body,
        out_shape=(
            jax.ShapeDtypeStruct((_META_R, _META_R), jnp.int32),
            jax.ShapeDtypeStruct((1, _BE_PAD), jnp.int32),
        ),
    )(eflat2d)


_DISP_CHUNK = 64
_DISP_TOK_PER_W = N_TOKENS // SC_WORKERS


def _dispatch_body(x_hbm, dest_hbm, xs_hbm, de_v, do_v, rows_v, sem_s):
    wid = lax.axis_index("s") * SC_CORES + lax.axis_index("c")
    base_t = wid * _DISP_TOK_PER_W

    def chunk(ci, carry):
        t0 = base_t + ci * _DISP_CHUNK
        pltpu.sync_copy(x_hbm.at[pl.ds(t0, _DISP_CHUNK)], rows_v)
        pltpu.sync_copy(dest_hbm.at[pl.ds(t0, _DISP_CHUNK)], de_v)
        pltpu.sync_copy(dest_hbm.at[pl.ds(N_TOKENS + t0, _DISP_CHUNK)], do_v)
        cp1 = pltpu.async_copy(rows_v, xs_hbm.at[de_v], sem_s)
        cp2 = pltpu.async_copy(rows_v, xs_hbm.at[do_v], sem_s)
        cp1.wait()
        cp2.wait()
        return carry

    lax.fori_loop(0, _DISP_TOK_PER_W // _DISP_CHUNK, chunk, 0)


def _dispatch(x, dest):
    mesh = plsc.VectorSubcoreMesh(core_axis_name="c", subcore_axis_name="s")
    return pl.kernel(
        _dispatch_body,
        out_type=jax.ShapeDtypeStruct((CAPACITY, D_MODEL), jnp.float32),
        mesh=mesh,
        scratch_types=[
            pltpu.VMEM((_DISP_CHUNK,), jnp.int32),
            pltpu.VMEM((_DISP_CHUNK,), jnp.int32),
            pltpu.VMEM((_DISP_CHUNK, D_MODEL), jnp.float32),
            pltpu.SemaphoreType.DMA,
        ],
    )(x, dest)


def _ffn_body(be_ref, xs_ref, w1_ref, w2_ref, w3_ref, out_ref):
    b = pl.program_id(0)
    f = pl.program_id(1)
    nv = be_ref[_BE_PAD - 1]

    @pl.when(b < nv)
    def _():
        x = xs_ref[...]
        g = jnp.dot(x, w1_ref[0], preferred_element_type=jnp.float32)
        v = jnp.dot(x, w2_ref[0], preferred_element_type=jnp.float32)
        h = (g * lax.logistic(g)) * v
        p = jnp.dot(h, w3_ref[0], preferred_element_type=jnp.float32)

        @pl.when(f == 0)
        def _():
            out_ref[...] = p

        @pl.when(f != 0)
        def _():
            out_ref[...] = out_ref[...] + p


def _serp(b, f):
    return jnp.where(b % 2 == 0, f, NF - 1 - f)


def _fidx(b, f, be_r):
    nv = be_r[_BE_PAD - 1]
    bb = jnp.minimum(b, nv - 1)
    fe = _serp(bb, jnp.where(b < nv, f, NF - 1))
    return be_r[bb], fe


def _ffn(be, xs, w1, w2, w3):
    def w12_map(b, f, be_r):
        e, fe = _fidx(b, f, be_r)
        return (e, 0, fe)

    def w3_map(b, f, be_r):
        e, fe = _fidx(b, f, be_r)
        return (e, fe, 0)

    def xs_map(b, f, be_r):
        return (jnp.minimum(b, be_r[_BE_PAD - 1] - 1), 0)

    grid_spec = pltpu.PrefetchScalarGridSpec(
        num_scalar_prefetch=1,
        grid=(NB, NF),
        in_specs=[
            pl.BlockSpec((B_ROWS, D_MODEL), xs_map),
            pl.BlockSpec((1, D_MODEL, F_TILE), w12_map),
            pl.BlockSpec((1, D_MODEL, F_TILE), w12_map),
            pl.BlockSpec((1, F_TILE, D_MODEL), w3_map),
        ],
        out_specs=pl.BlockSpec((B_ROWS, D_MODEL), lambda b, f, be_r: (b, 0)),
    )
    return pl.pallas_call(
        _ffn_body,
        grid_spec=grid_spec,
        out_shape=jax.ShapeDtypeStruct((CAPACITY, D_MODEL), jnp.float32),
        compiler_params=pltpu.CompilerParams(
            dimension_semantics=("arbitrary", "arbitrary")),
    )(be, xs, w1, w2, w3)


_COMB_CHUNK = 32
_TOK_PER_W = N_TOKENS // SC_WORKERS


def _gather2_body(y_hbm, dest_hbm, ye_hbm, yo_hbm,
                  de_v, do_v, re_v, ro_v, sem_g):
    wid = lax.axis_index("s") * SC_CORES + lax.axis_index("c")
    base_t = wid * _TOK_PER_W

    def chunk(ci, carry):
        t0 = base_t + ci * _COMB_CHUNK
        pltpu.sync_copy(dest_hbm.at[pl.ds(t0, _COMB_CHUNK)], de_v)
        pltpu.sync_copy(dest_hbm.at[pl.ds(N_TOKENS + t0, _COMB_CHUNK)], do_v)
        cp1 = pltpu.async_copy(y_hbm.at[de_v], re_v, sem_g)
        cp2 = pltpu.async_copy(y_hbm.at[do_v], ro_v, sem_g)
        cp1.wait()
        cp2.wait()
        pltpu.sync_copy(re_v, ye_hbm.at[pl.ds(t0, _COMB_CHUNK)])
        pltpu.sync_copy(ro_v, yo_hbm.at[pl.ds(t0, _COMB_CHUNK)])
        return carry

    lax.fori_loop(0, _TOK_PER_W // _COMB_CHUNK, chunk, 0)


def _gather2(y, dest):
    mesh = plsc.VectorSubcoreMesh(core_axis_name="c", subcore_axis_name="s")
    return pl.kernel(
        _gather2_body,
        out_type=(
            jax.ShapeDtypeStruct((N_TOKENS, D_MODEL), jnp.float32),
            jax.ShapeDtypeStruct((N_TOKENS, D_MODEL), jnp.float32),
        ),
        mesh=mesh,
        scratch_types=[
            pltpu.VMEM((_COMB_CHUNK,), jnp.int32),
            pltpu.VMEM((_COMB_CHUNK,), jnp.int32),
            pltpu.VMEM((_COMB_CHUNK, D_MODEL), jnp.float32),
            pltpu.VMEM((_COMB_CHUNK, D_MODEL), jnp.float32),
            pltpu.SemaphoreType.DMA,
        ],
    )(y, dest)


_WADD_B = 512


def _wadd_body(ye_ref, yo_ref, w_ref, out_ref):
    w = w_ref[...]
    out_ref[...] = ye_ref[...] * w[:, 0:1] + yo_ref[...] * w[:, 1:2]


def _wadd(ye, yo, wts):
    nb = N_TOKENS // _WADD_B
    return pl.pallas_call(
        _wadd_body,
        grid=(nb,),
        in_specs=[
            pl.BlockSpec((_WADD_B, D_MODEL), lambda b: (b, 0)),
            pl.BlockSpec((_WADD_B, D_MODEL), lambda b: (b, 0)),
            pl.BlockSpec((_WADD_B, TOP_K), lambda b: (b, 0)),
        ],
        out_specs=pl.BlockSpec((_WADD_B, D_MODEL), lambda b: (b, 0)),
        out_shape=jax.ShapeDtypeStruct((N_TOKENS, D_MODEL), jnp.float32),
        compiler_params=pltpu.CompilerParams(
            dimension_semantics=("arbitrary",)),
    )(ye, yo, wts)


def kernel(x, expert_indices, expert_weights, w1, w2, w3):
    eflat2d = expert_indices.astype(jnp.int32).T.reshape(_META_R, _META_R)
    dest2d, be_pad = _metadata(eflat2d)
    dest = dest2d.reshape(N_SLOTS)
    be = be_pad[0]
    xs = _dispatch(x, dest)
    y = _ffn(be, xs, w1, w2, w3)
    ye, yo = _gather2(y, dest)
    return _wadd(ye, yo, expert_weights)
